# baseline (speedup 1.0000x reference)
import functools

import jax
import jax.numpy as jnp
from jax import lax
from jax.experimental import pallas as pl
from jax.experimental.pallas import tpu as pltpu
from jax.experimental.pallas import tpu_sc as plsc

_C = 64


def _sort_body(x_ref, y_ref, p_ref, xi_ref, yi_ref):
    C, N = xi_ref.shape[1], xi_ref.shape[2]
    D = x_ref.shape[1]
    logn = N.bit_length() - 1
    idx_mask = jnp.int32(N - 1)
    key_mask = jnp.int32(-N)
    flip = jnp.int32(0x7FFFFFFF)

    p = p_ref[0]
    lane = lax.broadcasted_iota(jnp.int32, (C, N), 1)

    def make_key(t_ref):
        t = t_ref[0]
        k = jnp.zeros((C, N), jnp.float32)
        for d in range(D):
            k = k + p[:, d:d + 1] * t[d:d + 1, :]
        bi = lax.bitcast_convert_type(k, jnp.int32)
        bi = (bi & key_mask) | lane
        return jnp.where(bi < 0, bi ^ flip, bi)

    xu = make_key(x_ref)
    yu = make_key(y_ref)

    lane1 = lax.broadcasted_iota(jnp.int32, (1, N), 1)
    for kk in range(1, logn + 1):
        for j in range(kk - 1, -1, -1):
            s = 1 << j
            lo = (lane1 & s) == 0
            sel_min = (((lane1 >> j) ^ (lane1 >> kk)) & 1) == 0

            def cmpex(a):
                pa = jnp.where(lo, jnp.roll(a, -s, axis=1),
                               jnp.roll(a, s, axis=1))
                return jnp.where(sel_min, jnp.minimum(a, pa),
                                 jnp.maximum(a, pa))

            xu = cmpex(xu)
            yu = cmpex(yu)

    def extract(u):
        return jnp.where(u >= 0, u, ~u) & idx_mask

    xi_ref[0] = extract(xu)
    yi_ref[0] = extract(yu)


def _rank_indices(x_t, y_t, projections, interpret=False):
    B, D, N = x_t.shape
    L = projections.shape[1]
    C = _C if L % _C == 0 else L
    return pl.pallas_call(
        _sort_body,
        grid=(B, L // C),
        in_specs=[
            pl.BlockSpec((1, D, N), lambda b, l: (b, 0, 0)),
            pl.BlockSpec((1, D, N), lambda b, l: (b, 0, 0)),
            pl.BlockSpec((1, C, D), lambda b, l: (b, l, 0)),
        ],
        out_specs=[
            pl.BlockSpec((1, C, N), lambda b, l: (b, l, 0)),
            pl.BlockSpec((1, C, N), lambda b, l: (b, l, 0)),
        ],
        out_shape=[
            jax.ShapeDtypeStruct((B, L, N), jnp.int32),
            jax.ShapeDtypeStruct((B, L, N), jnp.int32),
        ],
        interpret=interpret,
    )(x_t, y_t, projections)


def _sc_pair_reduce(x_t, y_t, xidx, yidx):
    B, D, N = x_t.shape
    L = xidx.shape[1]
    info = plsc.get_sparse_core_info()
    NC, NS, LN = info.num_cores, info.num_subcores, info.num_lanes
    NW = NC * NS
    R = (B * L) // NW
    WPB = L // R
    RB = min(8, R)

    @functools.partial(
        pl.kernel,
        out_type=jax.ShapeDtypeStruct((NW * LN,), jnp.float32),
        mesh=plsc.VectorSubcoreMesh(core_axis_name="c", subcore_axis_name="s"),
        compiler_params=pltpu.CompilerParams(needs_layout_passes=False),
        scratch_types=(
            [pltpu.VMEM((N,), jnp.float32) for _ in range(2 * D)]
            + [
                pltpu.VMEM((RB * N,), jnp.int32),
                pltpu.VMEM((RB * N,), jnp.int32),
                pltpu.VMEM((LN,), jnp.float32),
            ]
        ),
    )
    def k(x_h, y_h, xi_h, yi_h, out_h, *scratch):
        tabs = scratch[:2 * D]
        xi_v, yi_v, acc_v = scratch[2 * D:]
        wid = lax.axis_index("s") * NC + lax.axis_index("c")
        b = wid // WPB
        l0 = (wid % WPB) * R
        for d in range(D):
            pltpu.sync_copy(x_h.at[pl.ds((b * D + d) * N, N)], tabs[d])
            pltpu.sync_copy(y_h.at[pl.ds((b * D + d) * N, N)], tabs[D + d])

        def rowblk(i, acc):
            base = (b * L + l0) * N + i * (RB * N)
            pltpu.sync_copy(xi_h.at[pl.ds(base, RB * N)], xi_v)
            pltpu.sync_copy(yi_h.at[pl.ds(base, RB * N)], yi_v)

            def chunk(c, a):
                xi = xi_v[pl.ds(c * LN, LN)]
                yi = yi_v[pl.ds(c * LN, LN)]
                for d in range(D):
                    xg = plsc.load_gather(tabs[d], [xi])
                    yg = plsc.load_gather(tabs[D + d], [yi])
                    df = xg - yg
                    a = a + df * df
                return a

            return lax.fori_loop(0, (RB * N) // LN, chunk, acc)

        acc = lax.fori_loop(0, R // RB, rowblk, jnp.zeros((LN,), jnp.float32))
        acc_v[...] = acc
        pltpu.sync_copy(acc_v, out_h.at[pl.ds(wid * LN, LN)])

    return k(x_t.reshape(-1), y_t.reshape(-1),
             xidx.reshape(-1), yidx.reshape(-1))


def kernel(x, y, projections):
    B, N, D = x.shape
    L = projections.shape[1]
    x_t = jnp.transpose(x, (0, 2, 1))
    y_t = jnp.transpose(y, (0, 2, 1))
    H = B // 2
    total = jnp.float32(0.0)
    for sl in (slice(0, H), slice(H, B)):
        xh, yh, ph = x_t[sl], y_t[sl], projections[sl]
        xidx, yidx = _rank_indices(xh, yh, ph)
        total = total + jnp.sum(_sc_pair_reduce(xh, yh, xidx, yidx))
    return (total / jnp.float32(B * L * N * D)).astype(jnp.float32)


# speedup vs baseline: 1.4750x; 1.4750x over previous
import functools

import jax
import jax.numpy as jnp
from jax import lax
from jax.experimental import pallas as pl
from jax.experimental.pallas import tpu as pltpu
from jax.experimental.pallas import tpu_sc as plsc

_C = 64


def _sort_body(x_ref, y_ref, p_ref, xi_ref, yi_ref):
    C, N = xi_ref.shape[1], xi_ref.shape[2]
    D = x_ref.shape[1]
    logn = N.bit_length() - 1
    idx_mask = jnp.int32(N - 1)
    key_mask = jnp.int32(-N)
    flip = jnp.int32(0x7FFFFFFF)

    p = p_ref[0]
    lane = lax.broadcasted_iota(jnp.int32, (C, N), 1)

    def make_key(t_ref):
        t = t_ref[0]
        k = jnp.zeros((C, N), jnp.float32)
        for d in range(D):
            k = k + p[:, d:d + 1] * t[d:d + 1, :]
        bi = lax.bitcast_convert_type(k, jnp.int32)
        bi = (bi & key_mask) | lane
        return jnp.where(bi < 0, bi ^ flip, bi)

    xu = make_key(x_ref)
    yu = make_key(y_ref)

    rot = 7 % logn

    def pbit(k):
        return (k + rot) % logn

    lane1 = lax.broadcasted_iota(jnp.int32, (1, N), 1)
    for kk in range(1, logn + 1):
        for j in range(kk - 1, -1, -1):
            pj = pbit(j)
            s = 1 << pj
            lo = (lane1 & s) == 0
            if kk == logn:
                sel_min = lo
            else:
                sel_min = (((lane1 >> pj) ^ (lane1 >> pbit(kk))) & 1) == 0

            def cmpex(a):
                pa = jnp.where(lo, jnp.roll(a, -s, axis=1),
                               jnp.roll(a, s, axis=1))
                return jnp.where(sel_min, jnp.minimum(a, pa),
                                 jnp.maximum(a, pa))

            xu = cmpex(xu)
            yu = cmpex(yu)

    def extract(u):
        return jnp.where(u >= 0, u, ~u) & idx_mask

    xi_ref[0] = extract(xu)
    yi_ref[0] = extract(yu)


def _rank_indices(x_t, y_t, projections, interpret=False):
    B, D, N = x_t.shape
    L = projections.shape[1]
    C = _C if L % _C == 0 else L
    return pl.pallas_call(
        _sort_body,
        grid=(B, L // C),
        in_specs=[
            pl.BlockSpec((1, D, N), lambda b, l: (b, 0, 0)),
            pl.BlockSpec((1, D, N), lambda b, l: (b, 0, 0)),
            pl.BlockSpec((1, C, D), lambda b, l: (b, l, 0)),
        ],
        out_specs=[
            pl.BlockSpec((1, C, N), lambda b, l: (b, l, 0)),
            pl.BlockSpec((1, C, N), lambda b, l: (b, l, 0)),
        ],
        out_shape=[
            jax.ShapeDtypeStruct((B, L, N), jnp.int32),
            jax.ShapeDtypeStruct((B, L, N), jnp.int32),
        ],
        interpret=interpret,
    )(x_t, y_t, projections)


def _sc_pair_reduce(x_t, y_t, xidx, yidx):
    B, D, N = x_t.shape
    L = xidx.shape[1]
    info = plsc.get_sparse_core_info()
    NC, NS, LN = info.num_cores, info.num_subcores, info.num_lanes
    NW = NC * NS
    R = (B * L) // NW
    WPB = L // R
    RB = min(8, R)

    @functools.partial(
        pl.kernel,
        out_type=jax.ShapeDtypeStruct((NW * LN,), jnp.float32),
        mesh=plsc.VectorSubcoreMesh(core_axis_name="c", subcore_axis_name="s"),
        compiler_params=pltpu.CompilerParams(needs_layout_passes=False),
        scratch_types=(
            [pltpu.VMEM((N,), jnp.float32) for _ in range(2 * D)]
            + [
                pltpu.VMEM((RB * N,), jnp.int32),
                pltpu.VMEM((RB * N,), jnp.int32),
                pltpu.VMEM((LN,), jnp.float32),
            ]
        ),
    )
    def k(x_h, y_h, xi_h, yi_h, out_h, *scratch):
        tabs = scratch[:2 * D]
        xi_v, yi_v, acc_v = scratch[2 * D:]
        wid = lax.axis_index("s") * NC + lax.axis_index("c")
        b = wid // WPB
        l0 = (wid % WPB) * R
        for d in range(D):
            pltpu.sync_copy(x_h.at[pl.ds((b * D + d) * N, N)], tabs[d])
            pltpu.sync_copy(y_h.at[pl.ds((b * D + d) * N, N)], tabs[D + d])

        def rowblk(i, acc):
            base = (b * L + l0) * N + i * (RB * N)
            pltpu.sync_copy(xi_h.at[pl.ds(base, RB * N)], xi_v)
            pltpu.sync_copy(yi_h.at[pl.ds(base, RB * N)], yi_v)

            def chunk(c, a):
                xi = xi_v[pl.ds(c * LN, LN)]
                yi = yi_v[pl.ds(c * LN, LN)]
                for d in range(D):
                    xg = plsc.load_gather(tabs[d], [xi])
                    yg = plsc.load_gather(tabs[D + d], [yi])
                    df = xg - yg
                    a = a + df * df
                return a

            return lax.fori_loop(0, (RB * N) // LN, chunk, acc)

        acc = lax.fori_loop(0, R // RB, rowblk, jnp.zeros((LN,), jnp.float32))
        acc_v[...] = acc
        pltpu.sync_copy(acc_v, out_h.at[pl.ds(wid * LN, LN)])

    return k(x_t.reshape(-1), y_t.reshape(-1),
             xidx.reshape(-1), yidx.reshape(-1))


def kernel(x, y, projections):
    B, N, D = x.shape
    L = projections.shape[1]
    x_t = jnp.transpose(x, (0, 2, 1))
    y_t = jnp.transpose(y, (0, 2, 1))
    xidx, yidx = _rank_indices(x_t, y_t, projections)
    partials = _sc_pair_reduce(x_t, y_t, xidx, yidx)
    return (jnp.sum(partials) / jnp.float32(B * L * N * D)).astype(jnp.float32)


# speedup vs baseline: 1.5959x; 1.0820x over previous
import functools

import jax
import jax.numpy as jnp
from jax import lax
from jax.experimental import pallas as pl
from jax.experimental.pallas import tpu as pltpu
from jax.experimental.pallas import tpu_sc as plsc

_C = 128


def _sort_body(x_ref, y_ref, p_ref, xi_ref, yi_ref):
    C, N = xi_ref.shape[1], xi_ref.shape[2]
    D = x_ref.shape[1]
    logn = N.bit_length() - 1
    idx_mask = jnp.int32(N - 1)
    key_mask = jnp.int32(-N)
    flip = jnp.int32(0x7FFFFFFF)

    p = p_ref[0]
    lane = lax.broadcasted_iota(jnp.int32, (C, N), 1)

    def make_key(t_ref):
        t = t_ref[0]
        k = jnp.zeros((C, N), jnp.float32)
        for d in range(D):
            k = k + p[:, d:d + 1] * t[d:d + 1, :]
        bi = lax.bitcast_convert_type(k, jnp.int32)
        bi = (bi & key_mask) | lane
        return jnp.where(bi < 0, bi ^ flip, bi)

    xu = make_key(x_ref)
    yu = make_key(y_ref)

    rot = 7 % logn

    def pbit(k):
        return (k + rot) % logn

    lane1 = lax.broadcasted_iota(jnp.int32, (1, N), 1)
    for kk in range(1, logn + 1):
        for j in range(kk - 1, -1, -1):
            pj = pbit(j)
            s = 1 << pj
            lo = (lane1 & s) == 0
            if kk == logn:
                sel_min = lo
            else:
                sel_min = (((lane1 >> pj) ^ (lane1 >> pbit(kk))) & 1) == 0

            def cmpex(a):
                pa = jnp.where(lo, jnp.roll(a, -s, axis=1),
                               jnp.roll(a, s, axis=1))
                return jnp.where(sel_min, jnp.minimum(a, pa),
                                 jnp.maximum(a, pa))

            xu = cmpex(xu)
            yu = cmpex(yu)

    def extract(u):
        return jnp.where(u >= 0, u, ~u) & idx_mask

    xi_ref[0] = extract(xu)
    yi_ref[0] = extract(yu)


def _rank_indices(x_t, y_t, projections, interpret=False):
    B, D, N = x_t.shape
    L = projections.shape[1]
    C = _C if L % _C == 0 else L
    return pl.pallas_call(
        _sort_body,
        grid=(B, L // C),
        in_specs=[
            pl.BlockSpec((1, D, N), lambda b, l: (b, 0, 0)),
            pl.BlockSpec((1, D, N), lambda b, l: (b, 0, 0)),
            pl.BlockSpec((1, C, D), lambda b, l: (b, l, 0)),
        ],
        out_specs=[
            pl.BlockSpec((1, C, N), lambda b, l: (b, l, 0)),
            pl.BlockSpec((1, C, N), lambda b, l: (b, l, 0)),
        ],
        out_shape=[
            jax.ShapeDtypeStruct((B, L, N), jnp.int32),
            jax.ShapeDtypeStruct((B, L, N), jnp.int32),
        ],
        interpret=interpret,
    )(x_t, y_t, projections)


def _sc_pair_reduce(x_t, y_t, xidx, yidx):
    B, D, N = x_t.shape
    L = xidx.shape[1]
    info = plsc.get_sparse_core_info()
    NC, NS, LN = info.num_cores, info.num_subcores, info.num_lanes
    NW = NC * NS
    R = (B * L) // NW
    WPB = L // R
    RB = min(8, R)

    @functools.partial(
        pl.kernel,
        out_type=jax.ShapeDtypeStruct((NW * LN,), jnp.float32),
        mesh=plsc.VectorSubcoreMesh(core_axis_name="c", subcore_axis_name="s"),
        compiler_params=pltpu.CompilerParams(needs_layout_passes=False),
        scratch_types=(
            [pltpu.VMEM((N,), jnp.float32) for _ in range(2 * D)]
            + [
                pltpu.VMEM((RB * N,), jnp.int32),
                pltpu.VMEM((RB * N,), jnp.int32),
                pltpu.VMEM((LN,), jnp.float32),
            ]
        ),
    )
    def k(x_h, y_h, xi_h, yi_h, out_h, *scratch):
        tabs = scratch[:2 * D]
        xi_v, yi_v, acc_v = scratch[2 * D:]
        wid = lax.axis_index("s") * NC + lax.axis_index("c")
        b = wid // WPB
        l0 = (wid % WPB) * R
        for d in range(D):
            pltpu.sync_copy(x_h.at[pl.ds((b * D + d) * N, N)], tabs[d])
            pltpu.sync_copy(y_h.at[pl.ds((b * D + d) * N, N)], tabs[D + d])

        def rowblk(i, acc):
            base = (b * L + l0) * N + i * (RB * N)
            pltpu.sync_copy(xi_h.at[pl.ds(base, RB * N)], xi_v)
            pltpu.sync_copy(yi_h.at[pl.ds(base, RB * N)], yi_v)

            def chunk(c, a):
                xi = xi_v[pl.ds(c * LN, LN)]
                yi = yi_v[pl.ds(c * LN, LN)]
                for d in range(D):
                    xg = plsc.load_gather(tabs[d], [xi])
                    yg = plsc.load_gather(tabs[D + d], [yi])
                    df = xg - yg
                    a = a + df * df
                return a

            return lax.fori_loop(0, (RB * N) // LN, chunk, acc)

        acc = lax.fori_loop(0, R // RB, rowblk, jnp.zeros((LN,), jnp.float32))
        acc_v[...] = acc
        pltpu.sync_copy(acc_v, out_h.at[pl.ds(wid * LN, LN)])

    return k(x_t.reshape(-1), y_t.reshape(-1),
             xidx.reshape(-1), yidx.reshape(-1))


def kernel(x, y, projections):
    B, N, D = x.shape
    L = projections.shape[1]
    x_t = jnp.transpose(x, (0, 2, 1))
    y_t = jnp.transpose(y, (0, 2, 1))
    xidx, yidx = _rank_indices(x_t, y_t, projections)
    partials = _sc_pair_reduce(x_t, y_t, xidx, yidx)
    return (jnp.sum(partials) / jnp.float32(B * L * N * D)).astype(jnp.float32)


# speedup vs baseline: 1.6431x; 1.0296x over previous
import functools

import jax
import jax.numpy as jnp
from jax import lax
from jax.experimental import pallas as pl
from jax.experimental.pallas import tpu as pltpu
from jax.experimental.pallas import tpu_sc as plsc

_C = 128


def _sort_body(x_ref, y_ref, p_ref, xi_ref, yi_ref):
    C = p_ref.shape[1]
    N = x_ref.shape[2]
    D = x_ref.shape[1]
    logn = N.bit_length() - 1
    idx_mask = jnp.int32(N - 1)
    key_mask = jnp.int32(-N)
    flip = jnp.int32(0x7FFFFFFF)

    p = p_ref[0]
    lane = lax.broadcasted_iota(jnp.int32, (C, N), 1)

    def make_key(t_ref):
        t = t_ref[0]
        k = jnp.zeros((C, N), jnp.float32)
        for d in range(D):
            k = k + p[:, d:d + 1] * t[d:d + 1, :]
        bi = lax.bitcast_convert_type(k, jnp.int32)
        bi = (bi & key_mask) | lane
        return jnp.where(bi < 0, bi ^ flip, bi)

    xu = make_key(x_ref)
    yu = make_key(y_ref)

    rot = 7 % logn

    def pbit(k):
        return (k + rot) % logn

    lane1 = lax.broadcasted_iota(jnp.int32, (1, N), 1)
    for kk in range(1, logn + 1):
        for j in range(kk - 1, -1, -1):
            pj = pbit(j)
            s = 1 << pj
            lo = (lane1 & s) == 0
            if kk == logn:
                sel_min = lo
            else:
                sel_min = (((lane1 >> pj) ^ (lane1 >> pbit(kk))) & 1) == 0

            def cmpex(a):
                pa = jnp.where(lo, jnp.roll(a, -s, axis=1),
                               jnp.roll(a, s, axis=1))
                return jnp.where(sel_min, jnp.minimum(a, pa),
                                 jnp.maximum(a, pa))

            xu = cmpex(xu)
            yu = cmpex(yu)

    def extract(u):
        return jnp.reshape(jnp.where(u >= 0, u, ~u) & idx_mask, (C * N,))

    xi_ref[...] = extract(xu)
    yi_ref[...] = extract(yu)


def _rank_indices(x_t, y_t, projections, interpret=False):
    B, D, N = x_t.shape
    L = projections.shape[1]
    C = _C if L % _C == 0 else L
    return pl.pallas_call(
        _sort_body,
        grid=(B, L // C),
        in_specs=[
            pl.BlockSpec((1, D, N), lambda b, l: (b, 0, 0)),
            pl.BlockSpec((1, D, N), lambda b, l: (b, 0, 0)),
            pl.BlockSpec((1, C, D), lambda b, l: (b, l, 0)),
        ],
        out_specs=[
            pl.BlockSpec((C * N,), lambda b, l: (b * (L // C) + l,)),
            pl.BlockSpec((C * N,), lambda b, l: (b * (L // C) + l,)),
        ],
        out_shape=[
            jax.ShapeDtypeStruct((B * L * N,), jnp.int32),
            jax.ShapeDtypeStruct((B * L * N,), jnp.int32),
        ],
        interpret=interpret,
    )(x_t, y_t, projections)


def _sc_pair_reduce(x_t, y_t, xidx, yidx):
    B, D, N = x_t.shape
    L = xidx.size // (B * N)
    info = plsc.get_sparse_core_info()
    NC, NS, LN = info.num_cores, info.num_subcores, info.num_lanes
    NW = NC * NS
    R = (B * L) // NW
    WPB = L // R
    RB = min(8, R)

    @functools.partial(
        pl.kernel,
        out_type=jax.ShapeDtypeStruct((NW * LN,), jnp.float32),
        mesh=plsc.VectorSubcoreMesh(core_axis_name="c", subcore_axis_name="s"),
        compiler_params=pltpu.CompilerParams(needs_layout_passes=False),
        scratch_types=(
            [pltpu.VMEM((N,), jnp.float32) for _ in range(2 * D)]
            + [
                pltpu.VMEM((RB * N,), jnp.int32),
                pltpu.VMEM((RB * N,), jnp.int32),
                pltpu.VMEM((LN,), jnp.float32),
            ]
        ),
    )
    def k(x_h, y_h, xi_h, yi_h, out_h, *scratch):
        tabs = scratch[:2 * D]
        xi_v, yi_v, acc_v = scratch[2 * D:]
        wid = lax.axis_index("s") * NC + lax.axis_index("c")
        b = wid // WPB
        l0 = (wid % WPB) * R
        for d in range(D):
            pltpu.sync_copy(x_h.at[pl.ds((b * D + d) * N, N)], tabs[d])
            pltpu.sync_copy(y_h.at[pl.ds((b * D + d) * N, N)], tabs[D + d])

        def rowblk(i, acc):
            base = (b * L + l0) * N + i * (RB * N)
            pltpu.sync_copy(xi_h.at[pl.ds(base, RB * N)], xi_v)
            pltpu.sync_copy(yi_h.at[pl.ds(base, RB * N)], yi_v)

            def chunk(c, a):
                xi = xi_v[pl.ds(c * LN, LN)]
                yi = yi_v[pl.ds(c * LN, LN)]
                for d in range(D):
                    xg = plsc.load_gather(tabs[d], [xi])
                    yg = plsc.load_gather(tabs[D + d], [yi])
                    df = xg - yg
                    a = a + df * df
                return a

            return lax.fori_loop(0, (RB * N) // LN, chunk, acc)

        acc = lax.fori_loop(0, R // RB, rowblk, jnp.zeros((LN,), jnp.float32))
        acc_v[...] = acc
        pltpu.sync_copy(acc_v, out_h.at[pl.ds(wid * LN, LN)])

    return k(x_t.reshape(-1), y_t.reshape(-1),
             xidx.reshape(-1), yidx.reshape(-1))


def kernel(x, y, projections):
    B, N, D = x.shape
    L = projections.shape[1]
    x_t = jnp.transpose(x, (0, 2, 1))
    y_t = jnp.transpose(y, (0, 2, 1))
    xidx, yidx = _rank_indices(x_t, y_t, projections)
    partials = _sc_pair_reduce(x_t, y_t, xidx, yidx)
    return (jnp.sum(partials) / jnp.float32(B * L * N * D)).astype(jnp.float32)


# speedup vs baseline: 1.6670x; 1.0145x over previous
import functools

import jax
import jax.numpy as jnp
from jax import lax
from jax.experimental import pallas as pl
from jax.experimental.pallas import tpu as pltpu
from jax.experimental.pallas import tpu_sc as plsc

_C = 128


def _sort_body(x_ref, y_ref, p_ref, xi_ref, yi_ref):
    C = p_ref.shape[1]
    N = x_ref.shape[2]
    D = x_ref.shape[1]
    logn = N.bit_length() - 1
    idx_mask = jnp.int32(N - 1)
    key_mask = jnp.int32(-N)
    flip = jnp.int32(0x7FFFFFFF)

    p = p_ref[0]
    lane = lax.broadcasted_iota(jnp.int32, (C, N), 1)

    def make_key(t_ref):
        t = t_ref[0]
        k = jnp.zeros((C, N), jnp.float32)
        for d in range(D):
            k = k + p[:, d:d + 1] * t[d:d + 1, :]
        bi = lax.bitcast_convert_type(k, jnp.int32)
        bi = (bi & key_mask) | lane
        return jnp.where(bi < 0, bi ^ flip, bi)

    xu = make_key(x_ref)
    yu = make_key(y_ref)

    rot = 7 % logn

    def pbit(k):
        return (k + rot) % logn

    lane1 = lax.broadcasted_iota(jnp.int32, (1, N), 1)
    for kk in range(1, logn + 1):
        for j in range(kk - 1, -1, -1):
            pj = pbit(j)
            s = 1 << pj
            lo = (lane1 & s) == 0
            if kk == logn:
                sel_min = lo
            else:
                sel_min = (((lane1 >> pj) ^ (lane1 >> pbit(kk))) & 1) == 0

            def cmpex(a):
                pa = jnp.where(lo, jnp.roll(a, -s, axis=1),
                               jnp.roll(a, s, axis=1))
                return jnp.where(sel_min, jnp.minimum(a, pa),
                                 jnp.maximum(a, pa))

            xu = cmpex(xu)
            yu = cmpex(yu)

    def extract(u):
        return jnp.reshape(jnp.where(u >= 0, u, ~u) & idx_mask, (C * N,))

    xi_ref[...] = extract(xu)
    yi_ref[...] = extract(yu)


def _rank_indices(x_t, y_t, projections, interpret=False):
    B, D, N = x_t.shape
    L = projections.shape[1]
    C = _C if L % _C == 0 else L
    return pl.pallas_call(
        _sort_body,
        grid=(B, L // C),
        in_specs=[
            pl.BlockSpec((1, D, N), lambda b, l: (b, 0, 0)),
            pl.BlockSpec((1, D, N), lambda b, l: (b, 0, 0)),
            pl.BlockSpec((1, C, D), lambda b, l: (b, l, 0)),
        ],
        out_specs=[
            pl.BlockSpec((C * N,), lambda b, l: (b * (L // C) + l,)),
            pl.BlockSpec((C * N,), lambda b, l: (b * (L // C) + l,)),
        ],
        out_shape=[
            jax.ShapeDtypeStruct((B * L * N,), jnp.int32),
            jax.ShapeDtypeStruct((B * L * N,), jnp.int32),
        ],
        interpret=interpret,
    )(x_t, y_t, projections)


def _pack_tables(x_t):
    b0 = lax.bitcast_convert_type(
        x_t[:, 0, :].astype(jnp.bfloat16), jnp.uint16).astype(jnp.uint32)
    b1 = lax.bitcast_convert_type(
        x_t[:, 1, :].astype(jnp.bfloat16), jnp.uint16).astype(jnp.uint32)
    packed = lax.bitcast_convert_type(b0 | (b1 << 16), jnp.float32)
    return jnp.stack([packed, x_t[:, 2, :]], axis=1).reshape(-1)


def _sc_pair_reduce(x_t, y_t, xidx, yidx):
    B, D, N = x_t.shape
    L = xidx.size // (B * N)
    info = plsc.get_sparse_core_info()
    NC, NS, LN = info.num_cores, info.num_subcores, info.num_lanes
    NW = NC * NS
    R = (B * L) // NW
    WPB = L // R
    RB = min(8, R)
    hi16 = jnp.int32(-65536)

    @functools.partial(
        pl.kernel,
        out_type=jax.ShapeDtypeStruct((NW * LN,), jnp.float32),
        mesh=plsc.VectorSubcoreMesh(core_axis_name="c", subcore_axis_name="s"),
        compiler_params=pltpu.CompilerParams(needs_layout_passes=False),
        scratch_types=[
            pltpu.VMEM((N,), jnp.float32),
            pltpu.VMEM((N,), jnp.float32),
            pltpu.VMEM((N,), jnp.float32),
            pltpu.VMEM((N,), jnp.float32),
            pltpu.VMEM((RB * N,), jnp.int32),
            pltpu.VMEM((RB * N,), jnp.int32),
            pltpu.VMEM((LN,), jnp.float32),
        ],
    )
    def k(x_h, y_h, xi_h, yi_h, out_h, xpk, x2t, ypk, y2t, xi_v, yi_v, acc_v):
        wid = lax.axis_index("s") * NC + lax.axis_index("c")
        b = wid // WPB
        l0 = (wid % WPB) * R
        pltpu.sync_copy(x_h.at[pl.ds((b * 2 + 0) * N, N)], xpk)
        pltpu.sync_copy(x_h.at[pl.ds((b * 2 + 1) * N, N)], x2t)
        pltpu.sync_copy(y_h.at[pl.ds((b * 2 + 0) * N, N)], ypk)
        pltpu.sync_copy(y_h.at[pl.ds((b * 2 + 1) * N, N)], y2t)

        def pair_term(xi, yi):
            gx = plsc.bitcast(plsc.load_gather(xpk, [xi]), jnp.int32)
            gy = plsc.bitcast(plsc.load_gather(ypk, [yi]), jnp.int32)
            d0 = (plsc.bitcast(gx << 16, jnp.float32)
                  - plsc.bitcast(gy << 16, jnp.float32))
            d1 = (plsc.bitcast(gx & hi16, jnp.float32)
                  - plsc.bitcast(gy & hi16, jnp.float32))
            d2 = plsc.load_gather(x2t, [xi]) - plsc.load_gather(y2t, [yi])
            return d0 * d0 + d1 * d1 + d2 * d2

        def rowblk(i, acc):
            base = (b * L + l0) * N + i * (RB * N)
            pltpu.sync_copy(xi_h.at[pl.ds(base, RB * N)], xi_v)
            pltpu.sync_copy(yi_h.at[pl.ds(base, RB * N)], yi_v)

            def chunk(c, a):
                for u in range(2):
                    off = (2 * c + u) * LN
                    xi = xi_v[pl.ds(off, LN)]
                    yi = yi_v[pl.ds(off, LN)]
                    a = a + pair_term(xi, yi)
                return a

            return lax.fori_loop(0, (RB * N) // (2 * LN), chunk, acc)

        acc = lax.fori_loop(0, R // RB, rowblk, jnp.zeros((LN,), jnp.float32))
        acc_v[...] = acc
        pltpu.sync_copy(acc_v, out_h.at[pl.ds(wid * LN, LN)])

    return k(_pack_tables(x_t), _pack_tables(y_t), xidx, yidx)


def kernel(x, y, projections):
    B, N, D = x.shape
    L = projections.shape[1]
    x_t = jnp.transpose(x, (0, 2, 1))
    y_t = jnp.transpose(y, (0, 2, 1))
    xidx, yidx = _rank_indices(x_t, y_t, projections)
    partials = _sc_pair_reduce(x_t, y_t, xidx, yidx)
    return (jnp.sum(partials) / jnp.float32(B * L * N * D)).astype(jnp.float32)


# speedup vs baseline: 1.6861x; 1.0115x over previous
import functools

import jax
import jax.numpy as jnp
from jax import lax
from jax.experimental import pallas as pl
from jax.experimental.pallas import tpu as pltpu
from jax.experimental.pallas import tpu_sc as plsc

_C = 128


def _sort_body(x_ref, y_ref, p_ref, xi_ref, yi_ref):
    C = p_ref.shape[1]
    N = x_ref.shape[2]
    D = x_ref.shape[1]
    logn = N.bit_length() - 1
    idx_mask = jnp.int32(N - 1)
    key_mask = jnp.int32(-N)
    flip = jnp.int32(0x7FFFFFFF)

    p = p_ref[0]
    lane = lax.broadcasted_iota(jnp.int32, (C, N), 1)

    def make_key(t_ref):
        t = t_ref[0]
        k = jnp.zeros((C, N), jnp.float32)
        for d in range(D):
            k = k + p[:, d:d + 1] * t[d:d + 1, :]
        bi = lax.bitcast_convert_type(k, jnp.int32)
        bi = (bi & key_mask) | lane
        return jnp.where(bi < 0, bi ^ flip, bi)

    xu = make_key(x_ref)
    yu = make_key(y_ref)

    rot = 7 % logn

    def pbit(k):
        return (k + rot) % logn

    lane1 = lax.broadcasted_iota(jnp.int32, (1, N), 1)
    for kk in range(1, logn + 1):
        for j in range(kk - 1, -1, -1):
            pj = pbit(j)
            s = 1 << pj
            lo = (lane1 & s) == 0
            if kk == logn:
                sel_min = lo
            else:
                sel_min = (((lane1 >> pj) ^ (lane1 >> pbit(kk))) & 1) == 0

            def cmpex(a):
                pa = jnp.where(lo, pltpu.roll(a, N - s, axis=1),
                               pltpu.roll(a, s, axis=1))
                return jnp.where(sel_min, jnp.minimum(a, pa),
                                 jnp.maximum(a, pa))

            xu = cmpex(xu)
            yu = cmpex(yu)

    def extract(u):
        return jnp.reshape(jnp.where(u >= 0, u, ~u) & idx_mask, (C * N,))

    xi_ref[...] = extract(xu)
    yi_ref[...] = extract(yu)


def _rank_indices(x_t, y_t, projections, interpret=False):
    B, D, N = x_t.shape
    L = projections.shape[1]
    C = _C if L % _C == 0 else L
    return pl.pallas_call(
        _sort_body,
        grid=(B, L // C),
        in_specs=[
            pl.BlockSpec((1, D, N), lambda b, l: (b, 0, 0)),
            pl.BlockSpec((1, D, N), lambda b, l: (b, 0, 0)),
            pl.BlockSpec((1, C, D), lambda b, l: (b, l, 0)),
        ],
        out_specs=[
            pl.BlockSpec((C * N,), lambda b, l: (b * (L // C) + l,)),
            pl.BlockSpec((C * N,), lambda b, l: (b * (L // C) + l,)),
        ],
        out_shape=[
            jax.ShapeDtypeStruct((B * L * N,), jnp.int32),
            jax.ShapeDtypeStruct((B * L * N,), jnp.int32),
        ],
        interpret=interpret,
    )(x_t, y_t, projections)


def _pack_tables(x_t):
    b0 = lax.bitcast_convert_type(
        x_t[:, 0, :].astype(jnp.bfloat16), jnp.uint16).astype(jnp.uint32)
    b1 = lax.bitcast_convert_type(
        x_t[:, 1, :].astype(jnp.bfloat16), jnp.uint16).astype(jnp.uint32)
    packed = lax.bitcast_convert_type(b0 | (b1 << 16), jnp.float32)
    return jnp.stack([packed, x_t[:, 2, :]], axis=1).reshape(-1)


def _sc_pair_reduce(x_t, y_t, xidx, yidx):
    B, D, N = x_t.shape
    L = xidx.size // (B * N)
    info = plsc.get_sparse_core_info()
    NC, NS, LN = info.num_cores, info.num_subcores, info.num_lanes
    NW = NC * NS
    R = (B * L) // NW
    WPB = L // R
    RB = min(8, R)
    hi16 = jnp.int32(-65536)

    @functools.partial(
        pl.kernel,
        out_type=jax.ShapeDtypeStruct((NW * LN,), jnp.float32),
        mesh=plsc.VectorSubcoreMesh(core_axis_name="c", subcore_axis_name="s"),
        compiler_params=pltpu.CompilerParams(needs_layout_passes=False),
        scratch_types=[
            pltpu.VMEM((N,), jnp.float32),
            pltpu.VMEM((N,), jnp.float32),
            pltpu.VMEM((N,), jnp.float32),
            pltpu.VMEM((N,), jnp.float32),
            pltpu.VMEM((RB * N,), jnp.int32),
            pltpu.VMEM((RB * N,), jnp.int32),
            pltpu.VMEM((LN,), jnp.float32),
        ],
    )
    def k(x_h, y_h, xi_h, yi_h, out_h, xpk, x2t, ypk, y2t, xi_v, yi_v, acc_v):
        wid = lax.axis_index("s") * NC + lax.axis_index("c")
        b = wid // WPB
        l0 = (wid % WPB) * R
        pltpu.sync_copy(x_h.at[pl.ds((b * 2 + 0) * N, N)], xpk)
        pltpu.sync_copy(x_h.at[pl.ds((b * 2 + 1) * N, N)], x2t)
        pltpu.sync_copy(y_h.at[pl.ds((b * 2 + 0) * N, N)], ypk)
        pltpu.sync_copy(y_h.at[pl.ds((b * 2 + 1) * N, N)], y2t)

        def pair_term(xi, yi):
            gx = plsc.bitcast(plsc.load_gather(xpk, [xi]), jnp.int32)
            gy = plsc.bitcast(plsc.load_gather(ypk, [yi]), jnp.int32)
            d0 = (plsc.bitcast(gx << 16, jnp.float32)
                  - plsc.bitcast(gy << 16, jnp.float32))
            d1 = (plsc.bitcast(gx & hi16, jnp.float32)
                  - plsc.bitcast(gy & hi16, jnp.float32))
            d2 = plsc.load_gather(x2t, [xi]) - plsc.load_gather(y2t, [yi])
            return d0 * d0 + d1 * d1 + d2 * d2

        def rowblk(i, acc):
            base = (b * L + l0) * N + i * (RB * N)
            pltpu.sync_copy(xi_h.at[pl.ds(base, RB * N)], xi_v)
            pltpu.sync_copy(yi_h.at[pl.ds(base, RB * N)], yi_v)

            def chunk(c, a):
                for u in range(2):
                    off = (2 * c + u) * LN
                    xi = xi_v[pl.ds(off, LN)]
                    yi = yi_v[pl.ds(off, LN)]
                    a = a + pair_term(xi, yi)
                return a

            return lax.fori_loop(0, (RB * N) // (2 * LN), chunk, acc)

        acc = lax.fori_loop(0, R // RB, rowblk, jnp.zeros((LN,), jnp.float32))
        acc_v[...] = acc
        pltpu.sync_copy(acc_v, out_h.at[pl.ds(wid * LN, LN)])

    return k(_pack_tables(x_t), _pack_tables(y_t), xidx, yidx)


def kernel(x, y, projections):
    B, N, D = x.shape
    L = projections.shape[1]
    x_t = jnp.transpose(x, (0, 2, 1))
    y_t = jnp.transpose(y, (0, 2, 1))
    xidx, yidx = _rank_indices(x_t, y_t, projections)
    partials = _sc_pair_reduce(x_t, y_t, xidx, yidx)
    return (jnp.sum(partials) / jnp.float32(B * L * N * D)).astype(jnp.float32)
